# Initial kernel scaffold; baseline (speedup 1.0000x reference)
import jax
import jax.numpy as jnp
from jax.experimental import pallas as pl


def kernel(pred1, proj2, pred2, proj1):
    raise NotImplementedError("write your pallas kernel here")


# speedup vs baseline: 1.6451x; 1.6451x over previous
import functools

import jax
import jax.numpy as jnp
from jax.experimental import pallas as pl
from jax.experimental.pallas import tpu as pltpu

N = 50000
D = 256
EPS = 1e-12
BLOCK = 2000


def _row_terms(a, b):
    na = jnp.sum(a * a, axis=1)
    nb = jnp.sum(b * b, axis=1)
    dab = jnp.sum(a * b, axis=1)
    sa = jnp.maximum(jnp.sqrt(na), EPS)
    sb = jnp.maximum(jnp.sqrt(nb), EPS)
    return jnp.sum(na / (sa * sa) + nb / (sb * sb) - 2.0 * dab / (sa * sb))


def _body(p1_ref, q2_ref, p2_ref, q1_ref, out_ref):
    i = pl.program_id(0)

    partial = _row_terms(p1_ref[...], q2_ref[...]) + _row_terms(
        p2_ref[...], q1_ref[...]
    )

    @pl.when(i == 0)
    def _():
        out_ref[0, 0] = 0.0

    out_ref[0, 0] += partial


@jax.jit
def kernel(pred1, proj2, pred2, proj1):
    grid = (N // BLOCK,)
    spec = pl.BlockSpec((BLOCK, D), lambda i: (i, 0))
    out = pl.pallas_call(
        _body,
        grid=grid,
        in_specs=[spec, spec, spec, spec],
        out_specs=pl.BlockSpec(
            (1, 1), lambda i: (0, 0), memory_space=pltpu.SMEM
        ),
        out_shape=jax.ShapeDtypeStruct((1, 1), jnp.float32),
    )(pred1, proj2, pred2, proj1)
    return out[0, 0] / 2.0
